# baseline (speedup 1.0000x reference)
import math

import jax
import jax.numpy as jnp
import numpy as np
from jax.experimental import pallas as pl
from jax.experimental.pallas import tpu as pltpu

ALPHA, GAMMA = 0.25, 2.0
HOR_POS_TH, HOR_NEG_TH, ROT_TH = 0.6, 0.4, 0.3
INV_NORM = np.array([10.0, 10.0, 5.0, 5.0, 10.0], dtype=np.float32)
M = 32
NF = 19
LANE = 128
DEG = math.pi / 180.0


def _gt_features(ann):
    cx, cy, w, h, th, cls_ = [ann[..., i] for i in range(6)]
    t = th * DEG
    a = jnp.sin(t) * 0.5
    b = jnp.cos(t) * 0.5
    x0 = cx - a * h - b * w
    y0 = cy + b * h - a * w
    x1 = cx + a * h - b * w
    y1 = cy - b * h - a * w
    x2 = 2.0 * cx - x0
    y2 = 2.0 * cy - y0
    x3 = 2.0 * cx - x1
    y3 = 2.0 * cy - y1
    xs = jnp.stack([x0, x1, x2, x3], 0)
    ys = jnp.stack([y0, y1, y2, y3], 0)
    hx1 = xs.min(0)
    hx2 = xs.max(0)
    hy1 = ys.min(0)
    hy2 = ys.max(0)
    area = (hx2 - hx1) * (hy2 - hy1)
    gw = jnp.maximum(w, 1.0)
    gh = jnp.maximum(h, 1.0)
    return jnp.stack([hx1, hy1, hx2, hy2, area,
                      x0, x1, x2, x3, y0, y1, y2, y3,
                      cx, cy, gw, gh, th, cls_], 1)


def _clip_stage(px, py, n, sf, n_out):
    n_in = len(px)
    sides = [sf(px[i], py[i]) for i in range(n_in)]
    em_x, em_y, em_c = [], [], []
    for i in range(n_in):
        active = n > i
        is_last = n == (i + 1)
        nx = jnp.where(is_last, px[0], px[(i + 1) % n_in])
        ny = jnp.where(is_last, py[0], py[(i + 1) % n_in])
        sn = jnp.where(is_last, sides[0], sides[(i + 1) % n_in])
        sc = sides[i]
        inc = sc >= 0.0
        inn = sn >= 0.0
        em_c.append(active & inc)
        em_x.append(px[i])
        em_y.append(py[i])
        denom = sc - sn
        tt = sc / jnp.where(denom == 0.0, 1.0, denom)
        em_c.append(active & (inc != inn))
        em_x.append(px[i] + tt * (nx - px[i]))
        em_y.append(py[i] + tt * (ny - py[i]))
    zero = jnp.zeros_like(px[0])
    ox = [zero] * n_out
    oy = [zero] * n_out
    pc = jnp.zeros_like(n)
    for e in range(2 * n_in):
        c = em_c[e]
        for s in range(n_out):
            hit = c & (pc == s)
            ox[s] = jnp.where(hit, em_x[e], ox[s])
            oy[s] = jnp.where(hit, em_y[e], oy[s])
        pc = pc + c.astype(n.dtype)
    return ox, oy, jnp.minimum(pc, n_out)


def _focal_kernel(gt_ref, cls_ref, reg_ref, anc_ref,
                  ocls_ref, oreg_ref, onp_ref, *, nc, s_blk, a_total):
    b = pl.program_id(0)
    j = pl.program_id(1)
    shape = (s_blk, LANE)

    ax1 = anc_ref[0]
    ay1 = anc_ref[1]
    ax2 = anc_ref[2]
    ay2 = anc_ref[3]
    ath = anc_ref[4]
    row_i = jax.lax.broadcasted_iota(jnp.int32, shape, 0)
    lane_i = jax.lax.broadcasted_iota(jnp.int32, shape, 1)
    aidx = (j * s_blk + row_i) * LANE + lane_i
    valid = aidx < a_total
    aw = jnp.where(valid, ax2 - ax1, 1.0)
    ah = jnp.where(valid, ay2 - ay1, 1.0)
    acx = ax1 + 0.5 * aw
    acy = ay1 + 0.5 * ah
    area_a = aw * ah

    base = b * (NF * M)

    def g(f, m):
        return gt_ref[base + f * M + m]

    hmax = jnp.full(shape, -1.0, jnp.float32)
    feats = [jnp.zeros(shape, jnp.float32) for _ in range(14)]
    for m in range(M):
        iw = jnp.maximum(jnp.minimum(ax2, g(2, m)) - jnp.maximum(ax1, g(0, m)), 0.0)
        ih = jnp.maximum(jnp.minimum(ay2, g(3, m)) - jnp.maximum(ay1, g(1, m)), 0.0)
        inter = iw * ih
        ua = jnp.maximum(area_a + g(4, m) - inter, 1e-8)
        iou = inter / ua
        better = iou > hmax
        hmax = jnp.where(better, iou, hmax)
        for f in range(14):
            feats[f] = jnp.where(better, g(5 + f, m), feats[f])
    qx = feats[0:4]
    qy = feats[4:8]
    gcx, gcy, gw, gh, gth, gcls = feats[8:14]

    n = jnp.full(shape, 4, jnp.int32)
    px, py = list(qx), list(qy)
    px, py, n = _clip_stage(px, py, n, lambda x, y: y - ay1, 5)
    px, py, n = _clip_stage(px, py, n, lambda x, y: ax2 - x, 6)
    px, py, n = _clip_stage(px, py, n, lambda x, y: ay2 - y, 7)
    px, py, n = _clip_stage(px, py, n, lambda x, y: x - ax1, 8)
    cr = jnp.zeros(shape, jnp.float32)
    for i in range(8):
        is_last = n == (i + 1)
        nx = jnp.where(is_last, px[0], px[(i + 1) % 8])
        ny = jnp.where(is_last, py[0], py[(i + 1) % 8])
        cr = cr + jnp.where(n > i, px[i] * ny - py[i] * nx, 0.0)
    inter_a = 0.5 * jnp.abs(cr)
    cr1 = jnp.zeros(shape, jnp.float32)
    for i in range(4):
        nx = qx[(i + 1) % 4]
        ny = qy[(i + 1) % 4]
        cr1 = cr1 + (qx[i] * ny - qy[i] * nx)
    a1 = 0.5 * jnp.abs(cr1)
    skew = inter_a / jnp.maximum(a1 + area_a - inter_a, 1e-8)

    pos = (hmax >= HOR_POS_TH) & (skew >= ROT_TH) & valid
    act = ((hmax < HOR_NEG_TH) | pos) & valid
    npos_row = jnp.sum(pos.astype(jnp.float32), axis=0)

    cls_acc = jnp.zeros((LANE,), jnp.float32)
    for c in range(nc):
        p = jnp.clip(cls_ref[0, c], 1e-4, 1.0 - 1e-4)
        isp = pos & (gcls == float(c))
        pos_term = ALPHA * (1.0 - p) * (1.0 - p) * (-jnp.log(p))
        neg_term = (1.0 - ALPHA) * p * p * (-jnp.log(1.0 - p))
        contrib = jnp.where(isp, pos_term, jnp.where(act, neg_term, 0.0))
        cls_acc = cls_acc + jnp.sum(contrib, axis=0)

    t5 = (
        (gcx - acx) / aw * INV_NORM[0],
        (gcy - acy) / ah * INV_NORM[1],
        jnp.log(gw / aw) * INV_NORM[2],
        jnp.log(gh / ah) * INV_NORM[3],
        (gth - ath) * DEG * INV_NORM[4],
    )
    reg_acc = jnp.zeros((LANE,), jnp.float32)
    for r in range(5):
        diff = jnp.abs(t5[r] - reg_ref[0, r])
        sl1 = jnp.where(diff <= 1.0 / 9.0, 4.5 * diff * diff, diff - 0.5 / 9.0)
        reg_acc = reg_acc + jnp.sum(jnp.where(pos, sl1, 0.0), axis=0)

    @pl.when(j == 0)
    def _():
        ocls_ref[0, 0] = jnp.zeros((LANE,), jnp.float32)
        oreg_ref[0, 0] = jnp.zeros((LANE,), jnp.float32)
        onp_ref[0, 0] = jnp.zeros((LANE,), jnp.float32)

    ocls_ref[0, 0] += cls_acc
    oreg_ref[0, 0] += reg_acc
    onp_ref[0, 0] += npos_row


def kernel(classifications, regressions, anchors, annotations):
    B, A, C = classifications.shape
    R = pl.cdiv(A, LANE)
    s_blk = R
    for s in (192, 96, 48, 40, 32, 24, 16, 8):
        if R > 56 and R % s == 0:
            s_blk = s
            break
    else:
        if R > 56:
            R = pl.cdiv(R, 8) * 8
            s_blk = 8
    apad = R * LANE
    nj = R // s_blk

    cls_t = jnp.transpose(classifications, (0, 2, 1))
    cls_t = jnp.pad(cls_t, ((0, 0), (0, 0), (0, apad - A)))
    cls_t = cls_t.reshape(B, C, R, LANE)
    reg_t = jnp.transpose(regressions, (0, 2, 1))
    reg_t = jnp.pad(reg_t, ((0, 0), (0, 0), (0, apad - A)))
    reg_t = reg_t.reshape(B, 5, R, LANE)
    anc = jnp.transpose(anchors[0], (1, 0))
    anc = jnp.pad(anc, ((0, 0), (0, apad - A))).reshape(5, R, LANE)
    gt = _gt_features(annotations).reshape(-1)

    import functools
    body = functools.partial(_focal_kernel, nc=C, s_blk=s_blk, a_total=A)
    out_sd = jax.ShapeDtypeStruct((B, 1, LANE), jnp.float32)
    outs = pl.pallas_call(
        body,
        grid=(B, nj),
        in_specs=[
            pl.BlockSpec(memory_space=pltpu.SMEM),
            pl.BlockSpec((1, C, s_blk, LANE), lambda b, j: (b, 0, j, 0)),
            pl.BlockSpec((1, 5, s_blk, LANE), lambda b, j: (b, 0, j, 0)),
            pl.BlockSpec((5, s_blk, LANE), lambda b, j: (0, j, 0)),
        ],
        out_specs=[
            pl.BlockSpec((1, 1, LANE), lambda b, j: (b, 0, 0)),
            pl.BlockSpec((1, 1, LANE), lambda b, j: (b, 0, 0)),
            pl.BlockSpec((1, 1, LANE), lambda b, j: (b, 0, 0)),
        ],
        out_shape=[out_sd, out_sd, out_sd],
        compiler_params=pltpu.CompilerParams(
            dimension_semantics=("parallel", "arbitrary")),
    )(gt, cls_t, reg_t, anc)

    cls_sum = outs[0].sum(axis=(1, 2))
    reg_sum = outs[1].sum(axis=(1, 2))
    npos = outs[2].sum(axis=(1, 2))
    cls_l = cls_sum / jnp.maximum(npos, 1.0)
    reg_l = jnp.where(npos > 0.0, reg_sum / jnp.maximum(npos * 5.0, 1.0), 0.0)
    return jnp.mean(cls_l, keepdims=True), jnp.mean(reg_l, keepdims=True)


# speedup vs baseline: 1.1556x; 1.1556x over previous
import math

import jax
import jax.numpy as jnp
import numpy as np
from jax.experimental import pallas as pl
from jax.experimental.pallas import tpu as pltpu

ALPHA, GAMMA = 0.25, 2.0
HOR_POS_TH, HOR_NEG_TH, ROT_TH = 0.6, 0.4, 0.3
INV_NORM = np.array([10.0, 10.0, 5.0, 5.0, 10.0], dtype=np.float32)
M = 32
NF = 19
LANE = 128
DEG = math.pi / 180.0


def _gt_features(ann):
    cx, cy, w, h, th, cls_ = [ann[..., i] for i in range(6)]
    t = th * DEG
    a = jnp.sin(t) * 0.5
    b = jnp.cos(t) * 0.5
    x0 = cx - a * h - b * w
    y0 = cy + b * h - a * w
    x1 = cx + a * h - b * w
    y1 = cy - b * h - a * w
    x2 = 2.0 * cx - x0
    y2 = 2.0 * cy - y0
    x3 = 2.0 * cx - x1
    y3 = 2.0 * cy - y1
    xs = jnp.stack([x0, x1, x2, x3], 0)
    ys = jnp.stack([y0, y1, y2, y3], 0)
    hx1 = xs.min(0)
    hx2 = xs.max(0)
    hy1 = ys.min(0)
    hy2 = ys.max(0)
    area = (hx2 - hx1) * (hy2 - hy1)
    gw = jnp.maximum(w, 1.0)
    gh = jnp.maximum(h, 1.0)
    return jnp.stack([hx1, hy1, hx2, hy2, area,
                      x0, x1, x2, x3, y0, y1, y2, y3,
                      cx, cy, gw, gh, th, cls_], 1)


def _clip_stage(px, py, n, sf, n_out):
    n_in = len(px)
    sides = [sf(px[i], py[i]) for i in range(n_in)]
    em_x, em_y, em_c = [], [], []
    for i in range(n_in):
        active = n > i
        is_last = n == (i + 1)
        nx = jnp.where(is_last, px[0], px[(i + 1) % n_in])
        ny = jnp.where(is_last, py[0], py[(i + 1) % n_in])
        sn = jnp.where(is_last, sides[0], sides[(i + 1) % n_in])
        sc = sides[i]
        inc = sc >= 0.0
        inn = sn >= 0.0
        em_c.append(active & inc)
        em_x.append(px[i])
        em_y.append(py[i])
        denom = sc - sn
        tt = sc / jnp.where(denom == 0.0, 1.0, denom)
        em_c.append(active & (inc != inn))
        em_x.append(px[i] + tt * (nx - px[i]))
        em_y.append(py[i] + tt * (ny - py[i]))
    zero = jnp.zeros_like(px[0])
    ox = [zero] * n_out
    oy = [zero] * n_out
    pc = jnp.zeros_like(n)
    for e in range(2 * n_in):
        c = em_c[e]
        for s in range(n_out):
            hit = c & (pc == s)
            ox[s] = jnp.where(hit, em_x[e], ox[s])
            oy[s] = jnp.where(hit, em_y[e], oy[s])
        pc = pc + c.astype(n.dtype)
    return ox, oy, jnp.minimum(pc, n_out)


def _focal_kernel(gt_ref, cls_ref, reg_ref, anc_ref,
                  ocls_ref, oreg_ref, onp_ref, *, nc, s_blk, a_total):
    b = pl.program_id(0)
    j = pl.program_id(1)
    shape = (s_blk, LANE)

    ax1 = anc_ref[0]
    ay1 = anc_ref[1]
    ax2 = anc_ref[2]
    ay2 = anc_ref[3]
    ath = anc_ref[4]
    row_i = jax.lax.broadcasted_iota(jnp.int32, shape, 0)
    lane_i = jax.lax.broadcasted_iota(jnp.int32, shape, 1)
    aidx = (j * s_blk + row_i) * LANE + lane_i
    valid = aidx < a_total
    aw = jnp.where(valid, ax2 - ax1, 1.0)
    ah = jnp.where(valid, ay2 - ay1, 1.0)
    acx = ax1 + 0.5 * aw
    acy = ay1 + 0.5 * ah
    area_a = aw * ah

    base = b * (NF * M)

    def g(f, m):
        return gt_ref[base + f * M + m]

    hmax = jnp.full(shape, -1.0, jnp.float32)
    feats = [jnp.zeros(shape, jnp.float32) for _ in range(14)]
    for m in range(M):
        iw = jnp.maximum(jnp.minimum(ax2, g(2, m)) - jnp.maximum(ax1, g(0, m)), 0.0)
        ih = jnp.maximum(jnp.minimum(ay2, g(3, m)) - jnp.maximum(ay1, g(1, m)), 0.0)
        inter = iw * ih
        ua = jnp.maximum(area_a + g(4, m) - inter, 1e-8)
        iou = inter / ua
        better = iou > hmax
        hmax = jnp.where(better, iou, hmax)
        for f in range(14):
            feats[f] = jnp.where(better, g(5 + f, m), feats[f])
    qx = feats[0:4]
    qy = feats[4:8]
    gcx, gcy, gw, gh, gth, gcls = feats[8:14]

    n = jnp.full(shape, 4, jnp.int32)
    px, py = list(qx), list(qy)
    px, py, n = _clip_stage(px, py, n, lambda x, y: y - ay1, 5)
    px, py, n = _clip_stage(px, py, n, lambda x, y: ax2 - x, 6)
    px, py, n = _clip_stage(px, py, n, lambda x, y: ay2 - y, 7)
    px, py, n = _clip_stage(px, py, n, lambda x, y: x - ax1, 8)
    cr = jnp.zeros(shape, jnp.float32)
    for i in range(8):
        is_last = n == (i + 1)
        nx = jnp.where(is_last, px[0], px[(i + 1) % 8])
        ny = jnp.where(is_last, py[0], py[(i + 1) % 8])
        cr = cr + jnp.where(n > i, px[i] * ny - py[i] * nx, 0.0)
    inter_a = 0.5 * jnp.abs(cr)
    cr1 = jnp.zeros(shape, jnp.float32)
    for i in range(4):
        nx = qx[(i + 1) % 4]
        ny = qy[(i + 1) % 4]
        cr1 = cr1 + (qx[i] * ny - qy[i] * nx)
    a1 = 0.5 * jnp.abs(cr1)
    skew = inter_a / jnp.maximum(a1 + area_a - inter_a, 1e-8)

    pos = (hmax >= HOR_POS_TH) & (skew >= ROT_TH) & valid
    act = ((hmax < HOR_NEG_TH) | pos) & valid
    npos_row = jnp.sum(pos.astype(jnp.float32), axis=0)

    cls_acc = jnp.zeros((LANE,), jnp.float32)
    for c in range(nc):
        p = jnp.clip(cls_ref[0, c], 1e-4, 1.0 - 1e-4)
        isp = pos & (gcls == float(c))
        pos_term = ALPHA * (1.0 - p) * (1.0 - p) * (-jnp.log(p))
        neg_term = (1.0 - ALPHA) * p * p * (-jnp.log(1.0 - p))
        contrib = jnp.where(isp, pos_term, jnp.where(act, neg_term, 0.0))
        cls_acc = cls_acc + jnp.sum(contrib, axis=0)

    t5 = (
        (gcx - acx) / aw * INV_NORM[0],
        (gcy - acy) / ah * INV_NORM[1],
        jnp.log(gw / aw) * INV_NORM[2],
        jnp.log(gh / ah) * INV_NORM[3],
        (gth - ath) * DEG * INV_NORM[4],
    )
    reg_acc = jnp.zeros((LANE,), jnp.float32)
    for r in range(5):
        diff = jnp.abs(t5[r] - reg_ref[0, r])
        sl1 = jnp.where(diff <= 1.0 / 9.0, 4.5 * diff * diff, diff - 0.5 / 9.0)
        reg_acc = reg_acc + jnp.sum(jnp.where(pos, sl1, 0.0), axis=0)

    @pl.when(j == 0)
    def _():
        ocls_ref[0, 0] = jnp.zeros((LANE,), jnp.float32)
        oreg_ref[0, 0] = jnp.zeros((LANE,), jnp.float32)
        onp_ref[0, 0] = jnp.zeros((LANE,), jnp.float32)

    ocls_ref[0, 0] += cls_acc
    oreg_ref[0, 0] += reg_acc
    onp_ref[0, 0] += npos_row


def kernel(classifications, regressions, anchors, annotations):
    B, A, C = classifications.shape
    R = pl.cdiv(A, LANE)
    s_blk = R
    for s in (24, 48, 40, 32, 16, 8):
        if R > 56 and R % s == 0:
            s_blk = s
            break
    else:
        if R > 56:
            R = pl.cdiv(R, 8) * 8
            s_blk = 8
    apad = R * LANE
    nj = R // s_blk

    cls_t = jnp.transpose(classifications, (0, 2, 1))
    cls_t = jnp.pad(cls_t, ((0, 0), (0, 0), (0, apad - A)))
    cls_t = cls_t.reshape(B, C, R, LANE)
    reg_t = jnp.transpose(regressions, (0, 2, 1))
    reg_t = jnp.pad(reg_t, ((0, 0), (0, 0), (0, apad - A)))
    reg_t = reg_t.reshape(B, 5, R, LANE)
    anc = jnp.transpose(anchors[0], (1, 0))
    anc = jnp.pad(anc, ((0, 0), (0, apad - A))).reshape(5, R, LANE)
    gt = _gt_features(annotations).reshape(-1)

    import functools
    body = functools.partial(_focal_kernel, nc=C, s_blk=s_blk, a_total=A)
    out_sd = jax.ShapeDtypeStruct((B, 1, LANE), jnp.float32)
    outs = pl.pallas_call(
        body,
        grid=(B, nj),
        in_specs=[
            pl.BlockSpec(memory_space=pltpu.SMEM),
            pl.BlockSpec((1, C, s_blk, LANE), lambda b, j: (b, 0, j, 0)),
            pl.BlockSpec((1, 5, s_blk, LANE), lambda b, j: (b, 0, j, 0)),
            pl.BlockSpec((5, s_blk, LANE), lambda b, j: (0, j, 0)),
        ],
        out_specs=[
            pl.BlockSpec((1, 1, LANE), lambda b, j: (b, 0, 0)),
            pl.BlockSpec((1, 1, LANE), lambda b, j: (b, 0, 0)),
            pl.BlockSpec((1, 1, LANE), lambda b, j: (b, 0, 0)),
        ],
        out_shape=[out_sd, out_sd, out_sd],
        compiler_params=pltpu.CompilerParams(
            dimension_semantics=("parallel", "arbitrary")),
    )(gt, cls_t, reg_t, anc)

    cls_sum = outs[0].sum(axis=(1, 2))
    reg_sum = outs[1].sum(axis=(1, 2))
    npos = outs[2].sum(axis=(1, 2))
    cls_l = cls_sum / jnp.maximum(npos, 1.0)
    reg_l = jnp.where(npos > 0.0, reg_sum / jnp.maximum(npos * 5.0, 1.0), 0.0)
    return jnp.mean(cls_l, keepdims=True), jnp.mean(reg_l, keepdims=True)
